# Initial kernel scaffold; baseline (speedup 1.0000x reference)
import jax
import jax.numpy as jnp
from jax.experimental import pallas as pl


def kernel(x, Wr, br, W1, b1, W2, b2):
    raise NotImplementedError("write your pallas kernel here")


# speedup vs baseline: 2.2817x; 2.2817x over previous
import functools
import math

import jax
import jax.numpy as jnp
from jax.experimental import pallas as pl
from jax.experimental.pallas import tpu as pltpu

D_MODEL = 1024
N_EXPERTS = 8
TOP_K = 2
D_FF = 4 * D_MODEL

BLK = 256
_SQRT_HALF = 1.0 / math.sqrt(2.0)


def _ffn_body(gid_ref, xs_ref, w1_ref, b1_ref, w2_ref, b2_ref, out_ref):
    xs = xs_ref[...]
    h = jnp.dot(xs, w1_ref[0], preferred_element_type=jnp.float32)
    h = h + b1_ref[0, 0].astype(jnp.float32)
    h = 0.5 * h * (1.0 + jax.lax.erf(h * _SQRT_HALF))
    y = jnp.dot(h.astype(w2_ref.dtype), w2_ref[0],
                preferred_element_type=jnp.float32)
    out_ref[...] = y + b2_ref[0, 0].astype(jnp.float32)


def _grouped_ffn(gid, xs, W1, b1, W2, b2, n_blocks, interpret=False):
    grid_spec = pltpu.PrefetchScalarGridSpec(
        num_scalar_prefetch=1,
        grid=(n_blocks,),
        in_specs=[
            pl.BlockSpec((BLK, D_MODEL), lambda i, gid: (i, 0)),
            pl.BlockSpec((1, D_MODEL, D_FF), lambda i, gid: (gid[i], 0, 0)),
            pl.BlockSpec((1, 1, D_FF), lambda i, gid: (gid[i], 0, 0)),
            pl.BlockSpec((1, D_FF, D_MODEL), lambda i, gid: (gid[i], 0, 0)),
            pl.BlockSpec((1, 1, D_MODEL), lambda i, gid: (gid[i], 0, 0)),
        ],
        out_specs=pl.BlockSpec((BLK, D_MODEL), lambda i, gid: (i, 0)),
    )
    return pl.pallas_call(
        _ffn_body,
        grid_spec=grid_spec,
        out_shape=jax.ShapeDtypeStruct((n_blocks * BLK, D_MODEL), jnp.float32),
        compiler_params=pltpu.CompilerParams(
            dimension_semantics=("arbitrary",)),
        interpret=interpret,
    )(gid, xs, W1, b1, W2, b2)


def kernel(x, Wr, br, W1, b1, W2, b2, interpret=False):
    B, L, D = x.shape
    xf = x.reshape(-1, D)
    N = xf.shape[0]
    A = N * TOP_K

    logits = xf @ Wr + br
    rw = jax.nn.softmax(logits, axis=-1)
    tkw, tki = jax.lax.top_k(rw, TOP_K)
    tkw = tkw / jnp.sum(tkw, axis=-1, keepdims=True)

    n_blocks = (A + N_EXPERTS * (BLK - 1) + BLK - 1) // BLK
    R = n_blocks * BLK

    e_flat = tki.reshape(-1).astype(jnp.int32)
    order = jnp.argsort(e_flat, stable=True)
    sorted_e = e_flat[order]
    counts = jnp.sum(jax.nn.one_hot(e_flat, N_EXPERTS, dtype=jnp.int32),
                     axis=0)
    pc = ((counts + BLK - 1) // BLK) * BLK
    cum_pc = jnp.cumsum(pc)
    pad_off = cum_pc - pc
    start = jnp.cumsum(counts) - counts
    k_ar = jnp.arange(A, dtype=jnp.int32)
    dest = pad_off[sorted_e] + (k_ar - start[sorted_e])
    tok = (order // TOP_K).astype(jnp.int32)

    gather_idx = jnp.zeros((R,), jnp.int32).at[dest].set(tok)
    xs = xf[gather_idx]
    pos = jnp.zeros((A,), jnp.int32).at[order].set(dest).reshape(N, TOP_K)
    gid = jnp.minimum(
        jnp.searchsorted(cum_pc, jnp.arange(n_blocks, dtype=jnp.int32) * BLK,
                         side='right'),
        N_EXPERTS - 1).astype(jnp.int32)

    bf = jnp.bfloat16
    ys = _grouped_ffn(gid, xs.astype(bf), W1.astype(bf),
                      b1.reshape(N_EXPERTS, 1, D_FF), W2.astype(bf),
                      b2.reshape(N_EXPERTS, 1, D_MODEL), n_blocks,
                      interpret=interpret)

    out = ys[pos[:, 0]] * tkw[:, :1] + ys[pos[:, 1]] * tkw[:, 1:]
    return out.reshape(B, L, D)
